# baseline (speedup 1.0000x reference)
import jax
import jax.numpy as jnp
from jax.experimental import pallas as pl
from jax.experimental.pallas import tpu as pltpu

_NEG = -1.0e30


def _fused_body(label_ref, xs_ref, xl_ref, w_ref, tl_ref, th_ref,
                probs_ref, hats_ref, loss_ref, lg_ref):
    b = pl.program_id(0)
    j = pl.program_id(1)
    nb = pl.num_programs(1)
    nbatch = pl.num_programs(0)

    w = w_ref[...]
    xs = xs_ref[0]
    xl = xl_ref[0]
    dn = (((1,), (0,)), ((), ()))
    ys = jax.lax.dot_general(xs, w, dn, preferred_element_type=jnp.float32)
    yl = jax.lax.dot_general(xl, w, dn, preferred_element_type=jnp.float32)

    tl = tl_ref[...]
    th = th_ref[...]
    tln = tl * jax.lax.rsqrt(jnp.sum(tl * tl, axis=1, keepdims=True))
    thn = th * jax.lax.rsqrt(jnp.sum(th * th, axis=1, keepdims=True))

    dt = (((1,), (1,)), ((), ()))
    rs = jax.lax.rsqrt(jnp.sum(ys * ys, axis=1, keepdims=True))
    rl = jax.lax.rsqrt(jnp.sum(yl * yl, axis=1, keepdims=True))
    zs = jax.lax.dot_general(tln, ys * rs, dt, preferred_element_type=jnp.float32)
    zl = jax.lax.dot_general(thn, yl * rl, dt, preferred_element_type=jnp.float32)
    lgT = zs + zl
    c = lgT.shape[0]
    bn = lgT.shape[1]
    pad = jnp.full((8 - c, bn), _NEG, jnp.float32)
    lg_ref[:, pl.ds(j * bn, bn)] = jnp.concatenate([lgT, pad], axis=0)

    @pl.when(j == nb - 1)
    def _tail():
        rem = lg_ref[...]
        cnt = jnp.zeros((8, 1), jnp.float32)
        s1 = jnp.zeros((8, 1), jnp.float32)
        s5 = jnp.zeros((8, 1), jnp.float32)
        s10 = jnp.zeros((8, 1), jnp.float32)
        for _ in range(10):
            v = jnp.max(rem, axis=1, keepdims=True)
            eq = rem == v
            m = jnp.sum(eq.astype(jnp.float32), axis=1, keepdims=True)
            s1 = s1 + v * jnp.clip(1.0 - cnt, 0.0, m)
            s5 = s5 + v * jnp.clip(5.0 - cnt, 0.0, m)
            s10 = s10 + v * jnp.clip(10.0 - cnt, 0.0, m)
            cnt = cnt + m
            rem = jnp.where(eq, -3.0e38, rem)
        p1 = s1
        p5 = s5 * (1.0 / 5.0)
        p10 = s10 * (1.0 / 10.0)
        pool = jnp.concatenate([p1, p5, p10, p1, p1, p1, p1, p1], axis=1)
        pmax = jnp.max(pool, axis=0, keepdims=True)
        ex = jnp.exp(pool - pmax)
        probs = ex / jnp.sum(ex, axis=0, keepdims=True)
        probs_ref[0] = probs
        ridx = jax.lax.broadcasted_iota(jnp.int32, (8, 8), 0)
        cand = jnp.where(pool == pmax, ridx, 8)
        hats_ref[0] = jnp.min(cand, axis=0, keepdims=True)
        m1 = jnp.max(p1, axis=0, keepdims=True)
        lse = jnp.log(jnp.sum(jnp.exp(p1 - m1), axis=0, keepdims=True)) + m1
        lab = label_ref[b]
        riota = jax.lax.broadcasted_iota(jnp.int32, (8, 1), 0)
        sel = jnp.sum(jnp.where(riota == lab, p1, 0.0), axis=0, keepdims=True)
        term = (lse - sel) / nbatch

        @pl.when(b == 0)
        def _init():
            loss_ref[...] = term

        @pl.when(b > 0)
        def _acc():
            loss_ref[...] = loss_ref[...] + term


def kernel(x_s, coord_s, x_l, coord_l, label, W_proj, text_low, text_high):
    B, N, D = x_s.shape
    E = W_proj.shape[1]
    C = text_low.shape[0]
    BN = 512
    NB = N // BN

    grid_spec = pltpu.PrefetchScalarGridSpec(
        num_scalar_prefetch=1,
        grid=(B, NB),
        in_specs=[
            pl.BlockSpec((1, BN, D), lambda b, j, *_: (b, j, 0)),
            pl.BlockSpec((1, BN, D), lambda b, j, *_: (b, j, 0)),
            pl.BlockSpec((D, E), lambda b, j, *_: (0, 0)),
            pl.BlockSpec((C, E), lambda b, j, *_: (0, 0)),
            pl.BlockSpec((C, E), lambda b, j, *_: (0, 0)),
        ],
        out_specs=[
            pl.BlockSpec((1, 8, 8), lambda b, j, *_: (b, 0, 0)),
            pl.BlockSpec((1, 1, 8), lambda b, j, *_: (b, 0, 0)),
            pl.BlockSpec((1, 1), lambda b, j, *_: (0, 0)),
        ],
        scratch_shapes=[pltpu.VMEM((8, N), jnp.float32)],
    )
    probs, hats, loss = pl.pallas_call(
        _fused_body,
        grid_spec=grid_spec,
        out_shape=[
            jax.ShapeDtypeStruct((B, 8, 8), jnp.float32),
            jax.ShapeDtypeStruct((B, 1, 8), jnp.int32),
            jax.ShapeDtypeStruct((1, 1), jnp.float32),
        ],
        compiler_params=pltpu.CompilerParams(
            dimension_semantics=("arbitrary", "arbitrary"),
        ),
    )(label, x_s, x_l, W_proj, text_low, text_high)

    Y_probs = jnp.transpose(probs[:, :C, :3], (2, 0, 1))
    Y_hats = jnp.transpose(hats[:, 0, :3], (1, 0))
    return (Y_probs, Y_hats, loss[0, 0])


# speedup vs baseline: 1.1957x; 1.1957x over previous
import jax
import jax.numpy as jnp
from jax.experimental import pallas as pl
from jax.experimental.pallas import tpu as pltpu

_NEG = -1.0e30


def _fused_body(label_ref, xs_ref, xl_ref, w_ref, tl_ref, th_ref,
                probs_ref, hats_ref, loss_ref, lg_ref):
    b = pl.program_id(0)
    j = pl.program_id(1)
    nb = pl.num_programs(1)
    nbatch = pl.num_programs(0)

    w = w_ref[...]
    xs = xs_ref[0]
    xl = xl_ref[0]
    dn = (((1,), (0,)), ((), ()))
    ys = jax.lax.dot_general(xs, w, dn, preferred_element_type=jnp.float32)
    yl = jax.lax.dot_general(xl, w, dn, preferred_element_type=jnp.float32)

    tl = tl_ref[...]
    th = th_ref[...]
    tln = tl * jax.lax.rsqrt(jnp.sum(tl * tl, axis=1, keepdims=True))
    thn = th * jax.lax.rsqrt(jnp.sum(th * th, axis=1, keepdims=True))

    dt = (((1,), (1,)), ((), ()))
    rs = jax.lax.rsqrt(jnp.sum(ys * ys, axis=1, keepdims=True))
    rl = jax.lax.rsqrt(jnp.sum(yl * yl, axis=1, keepdims=True))
    zs = jax.lax.dot_general(tln, ys * rs, dt, preferred_element_type=jnp.float32)
    zl = jax.lax.dot_general(thn, yl * rl, dt, preferred_element_type=jnp.float32)
    lgT = zs + zl
    c = lgT.shape[0]
    bn = lgT.shape[1]
    pad = jnp.full((8 - c, bn), _NEG, jnp.float32)
    lg_ref[:, pl.ds(j * bn, bn)] = jnp.concatenate([lgT, pad], axis=0)

    @pl.when(j == nb - 1)
    def _tail():
        rem = lg_ref[...]
        cnt = jnp.zeros((8, 1), jnp.float32)
        s1 = jnp.zeros((8, 1), jnp.float32)
        s5 = jnp.zeros((8, 1), jnp.float32)
        s10 = jnp.zeros((8, 1), jnp.float32)
        for _ in range(10):
            v = jnp.max(rem, axis=1, keepdims=True)
            eq = rem == v
            m = jnp.sum(eq.astype(jnp.float32), axis=1, keepdims=True)
            s1 = s1 + v * jnp.clip(1.0 - cnt, 0.0, m)
            s5 = s5 + v * jnp.clip(5.0 - cnt, 0.0, m)
            s10 = s10 + v * jnp.clip(10.0 - cnt, 0.0, m)
            cnt = cnt + m
            rem = jnp.where(eq, -3.0e38, rem)
        p1 = s1
        p5 = s5 * (1.0 / 5.0)
        p10 = s10 * (1.0 / 10.0)
        pool = jnp.concatenate([p1, p5, p10, p1, p1, p1, p1, p1], axis=1)
        pmax = jnp.max(pool, axis=0, keepdims=True)
        ex = jnp.exp(pool - pmax)
        probs = ex / jnp.sum(ex, axis=0, keepdims=True)
        probs_ref[0] = probs
        ridx = jax.lax.broadcasted_iota(jnp.int32, (8, 8), 0)
        cand = jnp.where(pool == pmax, ridx, 8)
        hats_ref[0] = jnp.min(cand, axis=0, keepdims=True)
        m1 = jnp.max(p1, axis=0, keepdims=True)
        lse = jnp.log(jnp.sum(jnp.exp(p1 - m1), axis=0, keepdims=True)) + m1
        lab = label_ref[b]
        riota = jax.lax.broadcasted_iota(jnp.int32, (8, 1), 0)
        sel = jnp.sum(jnp.where(riota == lab, p1, 0.0), axis=0, keepdims=True)
        term = (lse - sel) / nbatch

        @pl.when(b == 0)
        def _init():
            loss_ref[...] = term

        @pl.when(b > 0)
        def _acc():
            loss_ref[...] = loss_ref[...] + term


def kernel(x_s, coord_s, x_l, coord_l, label, W_proj, text_low, text_high):
    B, N, D = x_s.shape
    E = W_proj.shape[1]
    C = text_low.shape[0]
    BN = 1024
    NB = N // BN

    grid_spec = pltpu.PrefetchScalarGridSpec(
        num_scalar_prefetch=1,
        grid=(B, NB),
        in_specs=[
            pl.BlockSpec((1, BN, D), lambda b, j, *_: (b, j, 0)),
            pl.BlockSpec((1, BN, D), lambda b, j, *_: (b, j, 0)),
            pl.BlockSpec((D, E), lambda b, j, *_: (0, 0)),
            pl.BlockSpec((C, E), lambda b, j, *_: (0, 0)),
            pl.BlockSpec((C, E), lambda b, j, *_: (0, 0)),
        ],
        out_specs=[
            pl.BlockSpec((1, 8, 8), lambda b, j, *_: (b, 0, 0)),
            pl.BlockSpec((1, 1, 8), lambda b, j, *_: (b, 0, 0)),
            pl.BlockSpec((1, 1), lambda b, j, *_: (0, 0)),
        ],
        scratch_shapes=[pltpu.VMEM((8, N), jnp.float32)],
    )
    probs, hats, loss = pl.pallas_call(
        _fused_body,
        grid_spec=grid_spec,
        out_shape=[
            jax.ShapeDtypeStruct((B, 8, 8), jnp.float32),
            jax.ShapeDtypeStruct((B, 1, 8), jnp.int32),
            jax.ShapeDtypeStruct((1, 1), jnp.float32),
        ],
        compiler_params=pltpu.CompilerParams(
            dimension_semantics=("arbitrary", "arbitrary"),
        ),
    )(label, x_s, x_l, W_proj, text_low, text_high)

    Y_probs = jnp.transpose(probs[:, :C, :3], (2, 0, 1))
    Y_hats = jnp.transpose(hats[:, 0, :3], (1, 0))
    return (Y_probs, Y_hats, loss[0, 0])


# speedup vs baseline: 1.2186x; 1.0191x over previous
import jax
import jax.numpy as jnp
from jax.experimental import pallas as pl
from jax.experimental.pallas import tpu as pltpu

_NEG = -1.0e30


def _fused_body(label_ref, xs_ref, xl_ref, w_ref, tl_ref, th_ref,
                probs_ref, hats_ref, loss_ref, lg_ref):
    b = pl.program_id(0)
    j = pl.program_id(1)
    nb = pl.num_programs(1)
    nbatch = pl.num_programs(0)

    w = w_ref[...]
    xs = xs_ref[0]
    xl = xl_ref[0]
    dn = (((1,), (0,)), ((), ()))
    ys = jax.lax.dot_general(xs, w, dn, preferred_element_type=jnp.float32)
    yl = jax.lax.dot_general(xl, w, dn, preferred_element_type=jnp.float32)

    tl = tl_ref[...]
    th = th_ref[...]
    tln = tl * jax.lax.rsqrt(jnp.sum(tl * tl, axis=1, keepdims=True))
    thn = th * jax.lax.rsqrt(jnp.sum(th * th, axis=1, keepdims=True))

    dt = (((1,), (1,)), ((), ()))
    rs = jax.lax.rsqrt(jnp.sum(ys * ys, axis=1, keepdims=True))
    rl = jax.lax.rsqrt(jnp.sum(yl * yl, axis=1, keepdims=True))
    zs = jax.lax.dot_general(tln, ys * rs, dt, preferred_element_type=jnp.float32)
    zl = jax.lax.dot_general(thn, yl * rl, dt, preferred_element_type=jnp.float32)
    lgT = zs + zl
    c = lgT.shape[0]
    bn = lgT.shape[1]
    pad = jnp.full((8 - c, bn), _NEG, jnp.float32)
    lg_ref[:, pl.ds(j * bn, bn)] = jnp.concatenate([lgT, pad], axis=0)

    @pl.when(j == nb - 1)
    def _tail():
        rem = lg_ref[...]
        cnt = jnp.zeros((8, 1), jnp.float32)
        s1 = jnp.zeros((8, 1), jnp.float32)
        s5 = jnp.zeros((8, 1), jnp.float32)
        s10 = jnp.zeros((8, 1), jnp.float32)
        for _ in range(10):
            v = jnp.max(rem, axis=1, keepdims=True)
            eq = rem == v
            m = jnp.sum(eq.astype(jnp.float32), axis=1, keepdims=True)
            s1 = s1 + v * jnp.clip(1.0 - cnt, 0.0, m)
            s5 = s5 + v * jnp.clip(5.0 - cnt, 0.0, m)
            s10 = s10 + v * jnp.clip(10.0 - cnt, 0.0, m)
            cnt = cnt + m
            rem = jnp.where(eq, -3.0e38, rem)
        p1 = s1
        p5 = s5 * (1.0 / 5.0)
        p10 = s10 * (1.0 / 10.0)
        pool = jnp.concatenate([p1, p5, p10, p1, p1, p1, p1, p1], axis=1)
        pmax = jnp.max(pool, axis=0, keepdims=True)
        ex = jnp.exp(pool - pmax)
        probs = ex / jnp.sum(ex, axis=0, keepdims=True)
        probs_ref[0] = probs
        ridx = jax.lax.broadcasted_iota(jnp.int32, (8, 8), 0)
        cand = jnp.where(pool == pmax, ridx, 8)
        hats_ref[0] = jnp.min(cand, axis=0, keepdims=True)
        m1 = jnp.max(p1, axis=0, keepdims=True)
        lse = jnp.log(jnp.sum(jnp.exp(p1 - m1), axis=0, keepdims=True)) + m1
        lab = label_ref[b]
        riota = jax.lax.broadcasted_iota(jnp.int32, (8, 1), 0)
        sel = jnp.sum(jnp.where(riota == lab, p1, 0.0), axis=0, keepdims=True)
        term = (lse - sel) / nbatch

        @pl.when(b == 0)
        def _init():
            loss_ref[...] = term

        @pl.when(b > 0)
        def _acc():
            loss_ref[...] = loss_ref[...] + term


def kernel(x_s, coord_s, x_l, coord_l, label, W_proj, text_low, text_high):
    B, N, D = x_s.shape
    E = W_proj.shape[1]
    C = text_low.shape[0]
    BN = 2048
    NB = N // BN

    grid_spec = pltpu.PrefetchScalarGridSpec(
        num_scalar_prefetch=1,
        grid=(B, NB),
        in_specs=[
            pl.BlockSpec((1, BN, D), lambda b, j, *_: (b, j, 0)),
            pl.BlockSpec((1, BN, D), lambda b, j, *_: (b, j, 0)),
            pl.BlockSpec((D, E), lambda b, j, *_: (0, 0)),
            pl.BlockSpec((C, E), lambda b, j, *_: (0, 0)),
            pl.BlockSpec((C, E), lambda b, j, *_: (0, 0)),
        ],
        out_specs=[
            pl.BlockSpec((1, 8, 8), lambda b, j, *_: (b, 0, 0)),
            pl.BlockSpec((1, 1, 8), lambda b, j, *_: (b, 0, 0)),
            pl.BlockSpec((1, 1), lambda b, j, *_: (0, 0)),
        ],
        scratch_shapes=[pltpu.VMEM((8, N), jnp.float32)],
    )
    probs, hats, loss = pl.pallas_call(
        _fused_body,
        grid_spec=grid_spec,
        out_shape=[
            jax.ShapeDtypeStruct((B, 8, 8), jnp.float32),
            jax.ShapeDtypeStruct((B, 1, 8), jnp.int32),
            jax.ShapeDtypeStruct((1, 1), jnp.float32),
        ],
        compiler_params=pltpu.CompilerParams(
            dimension_semantics=("arbitrary", "arbitrary"),
        ),
    )(label, x_s, x_l, W_proj, text_low, text_high)

    Y_probs = jnp.transpose(probs[:, :C, :3], (2, 0, 1))
    Y_hats = jnp.transpose(hats[:, 0, :3], (1, 0))
    return (Y_probs, Y_hats, loss[0, 0])
